# baseline (speedup 1.0000x reference)
import functools

import jax
import jax.numpy as jnp
from jax import lax
from jax.experimental import pallas as pl
from jax.experimental.pallas import tpu as pltpu
from jax.experimental.pallas import tpu_sc as plsc

B, L = 1024, 20
DIM = 64
LANES = 16

NC, NS = 2, 16
NW = NC * NS
B_PER_W = B // NW
IDX_PER_W = B_PER_W * L
CHUNK = 128
N_CHUNK = IDX_PER_W // CHUNK

NV_BLK = 1024


def _sc_gather_sum(x_r, emb_table):
  @functools.partial(
      pl.kernel,
      out_type=jax.ShapeDtypeStruct((B, DIM), jnp.float32),
      mesh=plsc.VectorSubcoreMesh(core_axis_name="c", subcore_axis_name="s"),
      scratch_types=[
          pltpu.VMEM((N_CHUNK, CHUNK), jnp.int32),
          pltpu.VMEM((IDX_PER_W, DIM), jnp.float32),
          pltpu.VMEM((B_PER_W, DIM), jnp.float32),
          pltpu.SemaphoreType.DMA,
      ],
      compiler_params=pltpu.CompilerParams(use_tc_tiling_on_sc=False),
  )
  def k(x_hbm, tbl_hbm, out_hbm, idx_v, rows_v, out_v, sem):
    wid = lax.axis_index("s") * NC + lax.axis_index("c")
    pltpu.sync_copy(x_hbm.at[wid], idx_v)
    copies = [
        pltpu.async_copy(
            tbl_hbm.at[idx_v.at[c]], rows_v.at[pl.ds(c * CHUNK, CHUNK)], sem
        )
        for c in range(N_CHUNK)
    ]
    for cp in copies:
      cp.wait()

    def body(bi, carry):
      base = bi * L
      for d in range(DIM // LANES):
        acc = rows_v[base, pl.ds(d * LANES, LANES)]
        for j in range(1, L):
          acc = acc + rows_v[base + j, pl.ds(d * LANES, LANES)]
        out_v[bi, pl.ds(d * LANES, LANES)] = acc
      return carry

    lax.fori_loop(0, B_PER_W, body, 0)
    pltpu.sync_copy(out_v, out_hbm.at[pl.ds(wid * B_PER_W, B_PER_W)])

  return k(x_r, emb_table)


def _tc_project(summed, W, b):
  V = W.shape[0]
  grid = pl.cdiv(V, NV_BLK)

  def mm(s_ref, w_ref, b_ref, o_ref):
    o_ref[...] = (
        lax.dot_general(
            s_ref[...], w_ref[...], (((1,), (1,)), ((), ())),
            preferred_element_type=jnp.float32,
        )
        + b_ref[...]
    )

  return pl.pallas_call(
      mm,
      grid=(grid,),
      in_specs=[
          pl.BlockSpec((B, DIM), lambda i: (0, 0)),
          pl.BlockSpec((NV_BLK, DIM), lambda i: (i, 0)),
          pl.BlockSpec((1, NV_BLK), lambda i: (0, i)),
      ],
      out_specs=pl.BlockSpec((B, NV_BLK), lambda i: (0, i)),
      out_shape=jax.ShapeDtypeStruct((B, V), jnp.float32),
      compiler_params=pltpu.CompilerParams(
          dimension_semantics=("parallel",),
      ),
  )(summed, W, b.reshape(1, V))


@jax.jit
def kernel(X, emb_table, W, b):
  x_r = X.reshape(NW, N_CHUNK, CHUNK)
  summed = _sc_gather_sum(x_r, emb_table)
  return _tc_project(summed, W, b)


# speedup vs baseline: 2.4494x; 2.4494x over previous
import functools

import jax
import jax.numpy as jnp
from jax import lax
from jax.experimental import pallas as pl
from jax.experimental.pallas import tpu as pltpu
from jax.experimental.pallas import tpu_sc as plsc

B, L = 1024, 20
DIM = 64
LANES = 16

NC, NS = 2, 16
NW = NC * NS
B_PER_W = B // NW
IDX_PER_W = B_PER_W * L
CHUNK = 128
N_CHUNK = IDX_PER_W // CHUNK

NV_BLK = 1024


def _sc_gather_sum(x3, p16, tbl128):
  @functools.partial(
      pl.kernel,
      out_type=jax.ShapeDtypeStruct((B, DIM), jnp.float32),
      mesh=plsc.VectorSubcoreMesh(core_axis_name="c", subcore_axis_name="s"),
      scratch_types=[
          pltpu.VMEM((N_CHUNK, CHUNK), jnp.int32),
          pltpu.VMEM((IDX_PER_W * LANES,), jnp.float32),
          pltpu.VMEM((IDX_PER_W, 2 * DIM), jnp.float32),
          pltpu.VMEM((B_PER_W, DIM), jnp.float32),
          pltpu.SemaphoreType.DMA,
      ],
  )
  def k(x_hbm, p_hbm, tbl_hbm, out_hbm, idx_v, p_v, rows_v, out_v, sem):
    wid = lax.axis_index("s") * NC + lax.axis_index("c")
    pltpu.sync_copy(x_hbm.at[wid], idx_v)
    pltpu.sync_copy(
        p_hbm.at[pl.ds(wid * IDX_PER_W * LANES, IDX_PER_W * LANES)], p_v
    )
    copies = [
        pltpu.async_copy(
            tbl_hbm.at[idx_v.at[c]], rows_v.at[pl.ds(c * CHUNK, CHUNK)], sem
        )
        for c in range(N_CHUNK)
    ]
    for cp in copies:
      cp.wait()

    def body(bi, carry):
      base = bi * L
      accs = [jnp.zeros((LANES,), jnp.float32) for _ in range(DIM // LANES)]
      for j in range(L):
        r = base + j
        w = p_v[pl.ds(r * LANES, LANES)]
        for d in range(DIM // LANES):
          lo = rows_v[r, pl.ds(d * LANES, LANES)]
          hi = rows_v[r, pl.ds(DIM + d * LANES, LANES)]
          accs[d] = accs[d] + lo + w * (hi - lo)
      for d in range(DIM // LANES):
        out_v[bi, pl.ds(d * LANES, LANES)] = accs[d]
      return carry

    lax.fori_loop(0, B_PER_W, body, 0)
    pltpu.sync_copy(out_v, out_hbm.at[pl.ds(wid * B_PER_W, B_PER_W)])

  return k(x3, p16, tbl128)


def _tc_project_t(summed, w_t, b_row):
  V = w_t.shape[1]
  grid = pl.cdiv(V, NV_BLK)

  def mm(s_ref, w_ref, b_ref, o_ref):
    acc = lax.dot_general(
        w_ref[...], s_ref[...], (((0,), (1,)), ((), ())),
        preferred_element_type=jnp.float32,
    )
    o_ref[...] = acc + b_ref[...].T

  return pl.pallas_call(
      mm,
      grid=(grid,),
      in_specs=[
          pl.BlockSpec((B, DIM), lambda i: (0, 0)),
          pl.BlockSpec((DIM, NV_BLK), lambda i: (0, i)),
          pl.BlockSpec((1, NV_BLK), lambda i: (0, i)),
      ],
      out_specs=pl.BlockSpec((NV_BLK, B), lambda i: (i, 0)),
      out_shape=jax.ShapeDtypeStruct((V, B), jnp.float32),
      compiler_params=pltpu.CompilerParams(
          dimension_semantics=("parallel",),
      ),
  )(summed, w_t, b_row)


@jax.jit
def kernel(X, emb_table, W, b):
  xa = X.reshape(B * L)
  half_rows = jnp.right_shift(xa, 1)
  parity = jnp.bitwise_and(xa, 1).astype(jnp.float32)
  x3 = half_rows.reshape(NW, N_CHUNK, CHUNK)
  p16 = jnp.repeat(parity, LANES)
  tbl128 = emb_table.reshape(-1, 2 * DIM)
  summed = _sc_gather_sum(x3, p16, tbl128)
  out_t = _tc_project_t(summed, W.T, b.reshape(1, -1))
  return out_t.T


# speedup vs baseline: 2.6003x; 1.0616x over previous
import functools

import jax
import jax.numpy as jnp
from jax import lax
from jax.experimental import pallas as pl
from jax.experimental.pallas import tpu as pltpu
from jax.experimental.pallas import tpu_sc as plsc

B, L = 1024, 20
DIM = 64
LANES = 16

NC, NS = 2, 16
NW = NC * NS
B_PER_W = B // NW
IDX_PER_W = B_PER_W * L
CHUNK = 128
N_CHUNK = IDX_PER_W // CHUNK

NV_BLK = 1024
VHALF = 51200
R_BLK = 2048


def _tc_pack(t_t):
  grid = VHALF // R_BLK

  def pk(a_ref, b_ref, o_ref):
    o_ref[:, 0:DIM] = a_ref[...].T
    o_ref[:, DIM:2 * DIM] = b_ref[...].T

  return pl.pallas_call(
      pk,
      grid=(grid,),
      in_specs=[
          pl.BlockSpec((DIM, R_BLK), lambda i: (0, i)),
          pl.BlockSpec(
              (DIM, R_BLK),
              lambda i: (0, jnp.minimum(i + VHALF // R_BLK, 100000 // R_BLK)),
          ),
      ],
      out_specs=pl.BlockSpec((R_BLK, 2 * DIM), lambda i: (i, 0)),
      out_shape=jax.ShapeDtypeStruct((VHALF, 2 * DIM), jnp.float32),
      compiler_params=pltpu.CompilerParams(
          dimension_semantics=("parallel",),
      ),
  )(t_t, t_t)


def _sc_gather_sum(x3, p16, tbl128):
  @functools.partial(
      pl.kernel,
      out_type=jax.ShapeDtypeStruct((B, DIM), jnp.float32),
      mesh=plsc.VectorSubcoreMesh(core_axis_name="c", subcore_axis_name="s"),
      scratch_types=[
          pltpu.VMEM((N_CHUNK, CHUNK), jnp.int32),
          pltpu.VMEM((IDX_PER_W * LANES,), jnp.float32),
          pltpu.VMEM((IDX_PER_W, 2 * DIM), jnp.float32),
          pltpu.VMEM((B_PER_W, DIM), jnp.float32),
          pltpu.SemaphoreType.DMA,
      ],
  )
  def k(x_hbm, p_hbm, tbl_hbm, out_hbm, idx_v, p_v, rows_v, out_v, sem):
    wid = lax.axis_index("s") * NC + lax.axis_index("c")
    pltpu.sync_copy(x_hbm.at[wid], idx_v)
    pltpu.sync_copy(
        p_hbm.at[pl.ds(wid * IDX_PER_W * LANES, IDX_PER_W * LANES)], p_v
    )
    copies = [
        pltpu.async_copy(
            tbl_hbm.at[idx_v.at[c]], rows_v.at[pl.ds(c * CHUNK, CHUNK)], sem
        )
        for c in range(N_CHUNK)
    ]
    for cp in copies:
      cp.wait()

    def body(bi, carry):
      base = bi * L
      accs = [jnp.zeros((LANES,), jnp.float32) for _ in range(DIM // LANES)]
      for j in range(L):
        r = base + j
        w = p_v[pl.ds(r * LANES, LANES)]
        for d in range(DIM // LANES):
          lo = rows_v[r, pl.ds(d * LANES, LANES)]
          hi = rows_v[r, pl.ds(DIM + d * LANES, LANES)]
          accs[d] = accs[d] + lo + w * (hi - lo)
      for d in range(DIM // LANES):
        out_v[bi, pl.ds(d * LANES, LANES)] = accs[d]
      return carry

    lax.fori_loop(0, B_PER_W, body, 0)
    pltpu.sync_copy(out_v, out_hbm.at[pl.ds(wid * B_PER_W, B_PER_W)])

  return k(x3, p16, tbl128)


def _tc_project_t(summed, w_t, b_row):
  V = w_t.shape[1]
  grid = pl.cdiv(V, NV_BLK)

  def mm(s_ref, w_ref, b_ref, o_ref):
    acc = lax.dot_general(
        w_ref[...], s_ref[...], (((0,), (1,)), ((), ())),
        preferred_element_type=jnp.float32,
    )
    o_ref[...] = acc + b_ref[...].T

  return pl.pallas_call(
      mm,
      grid=(grid,),
      in_specs=[
          pl.BlockSpec((B, DIM), lambda i: (0, 0)),
          pl.BlockSpec((DIM, NV_BLK), lambda i: (0, i)),
          pl.BlockSpec((1, NV_BLK), lambda i: (0, i)),
      ],
      out_specs=pl.BlockSpec((NV_BLK, B), lambda i: (i, 0)),
      out_shape=jax.ShapeDtypeStruct((V, B), jnp.float32),
      compiler_params=pltpu.CompilerParams(
          dimension_semantics=("parallel",),
      ),
  )(summed, w_t, b_row)


@jax.jit
def kernel(X, emb_table, W, b):
  xa = X.reshape(B * L)
  in_hi = xa >= VHALF
  rows = jnp.where(in_hi, xa - VHALF, xa)
  parity = in_hi.astype(jnp.float32)
  x3 = rows.reshape(NW, N_CHUNK, CHUNK)
  p16 = jnp.repeat(parity, LANES)
  tbl128 = _tc_pack(emb_table.T)
  summed = _sc_gather_sum(x3, p16, tbl128)
  out_t = _tc_project_t(summed, W.T, b.reshape(1, -1))
  return out_t.T


# speedup vs baseline: 3.0484x; 1.1723x over previous
import functools

import jax
import jax.numpy as jnp
from jax import lax
from jax.experimental import pallas as pl
from jax.experimental.pallas import tpu as pltpu
from jax.experimental.pallas import tpu_sc as plsc

B, L = 1024, 20
DIM = 64
LANES = 16

NC, NS = 2, 16
NW = NC * NS
B_PER_W = B // NW
IDX_PER_W = B_PER_W * L
CHUNK = 128
N_CHUNK = IDX_PER_W // CHUNK

NV_BLK = 2048
VHALF = 51200
R_BLK = 6400


def _tc_pack(t_t):
  grid = VHALF // R_BLK

  def pk(a_ref, b_ref, o_ref):
    o_ref[:, 0:DIM] = a_ref[...].T
    o_ref[:, DIM:2 * DIM] = b_ref[...].T

  return pl.pallas_call(
      pk,
      grid=(grid,),
      in_specs=[
          pl.BlockSpec((DIM, R_BLK), lambda i: (0, i)),
          pl.BlockSpec(
              (DIM, R_BLK),
              lambda i: (0, jnp.minimum(i + VHALF // R_BLK, 100000 // R_BLK)),
          ),
      ],
      out_specs=pl.BlockSpec((R_BLK, 2 * DIM), lambda i: (i, 0)),
      out_shape=jax.ShapeDtypeStruct((VHALF, 2 * DIM), jnp.float32),
      compiler_params=pltpu.CompilerParams(
          dimension_semantics=("parallel",),
      ),
  )(t_t, t_t)


def _sc_gather_sum(x3, p16, tbl128):
  @functools.partial(
      pl.kernel,
      out_type=jax.ShapeDtypeStruct((B, DIM), jnp.float32),
      mesh=plsc.VectorSubcoreMesh(core_axis_name="c", subcore_axis_name="s"),
      scratch_types=[
          pltpu.VMEM((N_CHUNK, CHUNK), jnp.int32),
          pltpu.VMEM((IDX_PER_W * LANES,), jnp.float32),
          pltpu.VMEM((IDX_PER_W, 2 * DIM), jnp.float32),
          pltpu.VMEM((B_PER_W, DIM), jnp.float32),
          pltpu.SemaphoreType.DMA,
      ],
  )
  def k(x_hbm, p_hbm, tbl_hbm, out_hbm, idx_v, p_v, rows_v, out_v, sem):
    wid = lax.axis_index("s") * NC + lax.axis_index("c")
    pltpu.sync_copy(x_hbm.at[wid], idx_v)
    pltpu.sync_copy(
        p_hbm.at[pl.ds(wid * IDX_PER_W * LANES, IDX_PER_W * LANES)], p_v
    )
    copies = [
        pltpu.async_copy(
            tbl_hbm.at[idx_v.at[c]], rows_v.at[pl.ds(c * CHUNK, CHUNK)], sem
        )
        for c in range(N_CHUNK)
    ]
    for cp in copies:
      cp.wait()

    def body(bi, carry):
      base = bi * L
      accs = [jnp.zeros((LANES,), jnp.float32) for _ in range(DIM // LANES)]
      for j in range(L):
        r = base + j
        w = p_v[pl.ds(r * LANES, LANES)]
        for d in range(DIM // LANES):
          lo = rows_v[r, pl.ds(d * LANES, LANES)]
          hi = rows_v[r, pl.ds(DIM + d * LANES, LANES)]
          accs[d] = accs[d] + lo + w * (hi - lo)
      for d in range(DIM // LANES):
        out_v[bi, pl.ds(d * LANES, LANES)] = accs[d]
      return carry

    lax.fori_loop(0, B_PER_W, body, 0)
    pltpu.sync_copy(out_v, out_hbm.at[pl.ds(wid * B_PER_W, B_PER_W)])

  return k(x3, p16, tbl128)


def _tc_project_t(summed, w_t, b_row):
  V = w_t.shape[1]
  grid = pl.cdiv(V, NV_BLK)

  def mm(s_ref, w_ref, b_ref, o_ref):
    acc = lax.dot_general(
        w_ref[...], s_ref[...], (((0,), (1,)), ((), ())),
        preferred_element_type=jnp.float32,
    )
    o_ref[...] = acc + b_ref[...].T

  return pl.pallas_call(
      mm,
      grid=(grid,),
      in_specs=[
          pl.BlockSpec((B, DIM), lambda i: (0, 0)),
          pl.BlockSpec((DIM, NV_BLK), lambda i: (0, i)),
          pl.BlockSpec((1, NV_BLK), lambda i: (0, i)),
      ],
      out_specs=pl.BlockSpec((NV_BLK, B), lambda i: (i, 0)),
      out_shape=jax.ShapeDtypeStruct((V, B), jnp.float32),
      compiler_params=pltpu.CompilerParams(
          dimension_semantics=("parallel",),
      ),
  )(summed, w_t, b_row)


@jax.jit
def kernel(X, emb_table, W, b):
  xa = X.reshape(B * L)
  in_hi = xa >= VHALF
  rows = jnp.where(in_hi, xa - VHALF, xa)
  parity = in_hi.astype(jnp.float32)
  x3 = rows.reshape(NW, N_CHUNK, CHUNK)
  p16 = jnp.repeat(parity, LANES)
  tbl128 = _tc_pack(emb_table.T)
  summed = _sc_gather_sum(x3, p16, tbl128)
  out_t = _tc_project_t(summed, W.T, b.reshape(1, -1))
  return out_t.T


# speedup vs baseline: 3.3896x; 1.1119x over previous
import functools

import jax
import jax.numpy as jnp
from jax import lax
from jax.experimental import pallas as pl
from jax.experimental.pallas import tpu as pltpu
from jax.experimental.pallas import tpu_sc as plsc

B, L = 1024, 20
DIM = 64
LANES = 16

NC, NS = 2, 16
NW = NC * NS
B_PER_W = B // NW
IDX_PER_W = B_PER_W * L
CHUNK = 128
N_CHUNK = IDX_PER_W // CHUNK

NV_BLK = 2048
VHALF = 51200
R_BLK = 6400


def _tc_pack(t_t):
  grid = VHALF // R_BLK

  def pk(a_ref, b_ref, o_ref):
    o_ref[:, 0:DIM] = a_ref[...].T
    o_ref[:, DIM:2 * DIM] = b_ref[...].T

  return pl.pallas_call(
      pk,
      grid=(grid,),
      in_specs=[
          pl.BlockSpec((DIM, R_BLK), lambda i: (0, i)),
          pl.BlockSpec(
              (DIM, R_BLK),
              lambda i: (0, jnp.minimum(i + VHALF // R_BLK, 100000 // R_BLK)),
          ),
      ],
      out_specs=pl.BlockSpec((R_BLK, 2 * DIM), lambda i: (i, 0)),
      out_shape=jax.ShapeDtypeStruct((VHALF, 2 * DIM), jnp.float32),
      compiler_params=pltpu.CompilerParams(
          dimension_semantics=("parallel",),
      ),
  )(t_t, t_t)


def _sc_gather_sum(x1d, tbl128):
  @functools.partial(
      pl.kernel,
      out_type=jax.ShapeDtypeStruct((B, DIM), jnp.float32),
      mesh=plsc.VectorSubcoreMesh(core_axis_name="c", subcore_axis_name="s"),
      scratch_types=[
          pltpu.VMEM((IDX_PER_W + LANES,), jnp.int32),
          pltpu.VMEM((N_CHUNK, CHUNK), jnp.int32),
          pltpu.VMEM((IDX_PER_W, 2 * DIM), jnp.float32),
          pltpu.VMEM((B_PER_W, DIM), jnp.float32),
          pltpu.SemaphoreType.DMA,
      ],
  )
  def k(x_hbm, tbl_hbm, out_hbm, xraw_v, idx_v, rows_v, out_v, sem):
    wid = lax.axis_index("s") * NC + lax.axis_index("c")
    pltpu.sync_copy(x_hbm.at[pl.ds(wid * IDX_PER_W, IDX_PER_W)],
                    xraw_v.at[pl.ds(0, IDX_PER_W)])
    for c in range(N_CHUNK):
      for i in range(CHUNK // LANES):
        v = xraw_v[pl.ds((c * CHUNK + i * LANES), LANES)]
        idx_v[c, pl.ds(i * LANES, LANES)] = jnp.where(v >= VHALF, v - VHALF, v)
    copies = [
        pltpu.async_copy(
            tbl_hbm.at[idx_v.at[c]], rows_v.at[pl.ds(c * CHUNK, CHUNK)], sem
        )
        for c in range(N_CHUNK)
    ]
    for cp in copies:
      cp.wait()

    def body(bi, carry):
      base = bi * L
      accs = [jnp.zeros((LANES,), jnp.float32) for _ in range(DIM // LANES)]
      for j in range(L):
        r = base + j
        xs = xraw_v[pl.ds(r, LANES)][0]
        off = jnp.where(xs >= VHALF, DIM, 0)
        for d in range(DIM // LANES):
          accs[d] = accs[d] + rows_v[r, pl.ds(off + d * LANES, LANES)]
      for d in range(DIM // LANES):
        out_v[bi, pl.ds(d * LANES, LANES)] = accs[d]
      return carry

    lax.fori_loop(0, B_PER_W, body, 0)
    pltpu.sync_copy(out_v, out_hbm.at[pl.ds(wid * B_PER_W, B_PER_W)])

  return k(x1d, tbl128)


def _tc_project_t(summed, w_t, b_row):
  V = w_t.shape[1]
  grid = pl.cdiv(V, NV_BLK)

  def mm(s_ref, w_ref, b_ref, o_ref):
    acc = lax.dot_general(
        w_ref[...], s_ref[...], (((0,), (1,)), ((), ())),
        preferred_element_type=jnp.float32,
    )
    o_ref[...] = acc + b_ref[...].T

  return pl.pallas_call(
      mm,
      grid=(grid,),
      in_specs=[
          pl.BlockSpec((B, DIM), lambda i: (0, 0)),
          pl.BlockSpec((DIM, NV_BLK), lambda i: (0, i)),
          pl.BlockSpec((1, NV_BLK), lambda i: (0, i)),
      ],
      out_specs=pl.BlockSpec((NV_BLK, B), lambda i: (i, 0)),
      out_shape=jax.ShapeDtypeStruct((V, B), jnp.float32),
      compiler_params=pltpu.CompilerParams(
          dimension_semantics=("parallel",),
      ),
  )(summed, w_t, b_row)


@jax.jit
def kernel(X, emb_table, W, b):
  xa = X.reshape(B * L)
  tbl128 = _tc_pack(emb_table.T)
  summed = _sc_gather_sum(xa, tbl128)
  out_t = _tc_project_t(summed, W.T, b.reshape(1, -1))
  return out_t.T


# speedup vs baseline: 3.4383x; 1.0144x over previous
import functools

import jax
import jax.numpy as jnp
from jax import lax
from jax.experimental import pallas as pl
from jax.experimental.pallas import tpu as pltpu
from jax.experimental.pallas import tpu_sc as plsc

B, L = 1024, 20
DIM = 64
LANES = 16

NC, NS = 2, 16
NW = NC * NS
B_PER_W = B // NW
IDX_PER_W = B_PER_W * L
CHUNK = 128
N_CHUNK = IDX_PER_W // CHUNK

NV_BLK = 4096
VHALF = 51200
R_BLK = 12800


def _tc_pack(t_t):
  grid = VHALF // R_BLK

  def pk(a_ref, b_ref, o_ref):
    o_ref[:, 0:DIM] = a_ref[...].T
    o_ref[:, DIM:2 * DIM] = b_ref[...].T

  return pl.pallas_call(
      pk,
      grid=(grid,),
      in_specs=[
          pl.BlockSpec((DIM, R_BLK), lambda i: (0, i)),
          pl.BlockSpec(
              (DIM, R_BLK),
              lambda i: (0, jnp.minimum(i + VHALF // R_BLK, 100000 // R_BLK)),
          ),
      ],
      out_specs=pl.BlockSpec((R_BLK, 2 * DIM), lambda i: (i, 0)),
      out_shape=jax.ShapeDtypeStruct((VHALF, 2 * DIM), jnp.float32),
      compiler_params=pltpu.CompilerParams(
          dimension_semantics=("parallel",),
      ),
  )(t_t, t_t)


def _sc_gather_sum(x1d, tbl128):
  @functools.partial(
      pl.kernel,
      out_type=jax.ShapeDtypeStruct((B, DIM), jnp.float32),
      mesh=plsc.VectorSubcoreMesh(core_axis_name="c", subcore_axis_name="s"),
      scratch_types=[
          pltpu.VMEM((IDX_PER_W + LANES,), jnp.int32),
          pltpu.VMEM((N_CHUNK, CHUNK), jnp.int32),
          pltpu.VMEM((IDX_PER_W, 2 * DIM), jnp.float32),
          pltpu.VMEM((B_PER_W, DIM), jnp.float32),
          pltpu.SemaphoreType.DMA,
      ],
  )
  def k(x_hbm, tbl_hbm, out_hbm, xraw_v, idx_v, rows_v, out_v, sem):
    wid = lax.axis_index("s") * NC + lax.axis_index("c")
    pltpu.sync_copy(x_hbm.at[pl.ds(wid * IDX_PER_W, IDX_PER_W)],
                    xraw_v.at[pl.ds(0, IDX_PER_W)])
    for c in range(N_CHUNK):
      for i in range(CHUNK // LANES):
        v = xraw_v[pl.ds((c * CHUNK + i * LANES), LANES)]
        idx_v[c, pl.ds(i * LANES, LANES)] = jnp.where(v >= VHALF, v - VHALF, v)
    copies = [
        pltpu.async_copy(
            tbl_hbm.at[idx_v.at[c]], rows_v.at[pl.ds(c * CHUNK, CHUNK)], sem
        )
        for c in range(N_CHUNK)
    ]
    for cp in copies:
      cp.wait()

    def body(bi, carry):
      base = bi * L
      accs = [jnp.zeros((LANES,), jnp.float32) for _ in range(DIM // LANES)]
      for j in range(L):
        r = base + j
        xs = xraw_v[pl.ds(r, LANES)][0]
        off = jnp.where(xs >= VHALF, DIM, 0)
        for d in range(DIM // LANES):
          accs[d] = accs[d] + rows_v[r, pl.ds(off + d * LANES, LANES)]
      for d in range(DIM // LANES):
        out_v[bi, pl.ds(d * LANES, LANES)] = accs[d]
      return carry

    lax.fori_loop(0, B_PER_W, body, 0)
    pltpu.sync_copy(out_v, out_hbm.at[pl.ds(wid * B_PER_W, B_PER_W)])

  return k(x1d, tbl128)


def _tc_project_t(summed, w_t, b_row):
  V = w_t.shape[1]
  grid = pl.cdiv(V, NV_BLK)

  def mm(s_ref, w_ref, b_ref, o_ref):
    acc = lax.dot_general(
        w_ref[...], s_ref[...], (((0,), (1,)), ((), ())),
        preferred_element_type=jnp.float32,
    )
    o_ref[...] = acc + b_ref[...].T

  return pl.pallas_call(
      mm,
      grid=(grid,),
      in_specs=[
          pl.BlockSpec((B, DIM), lambda i: (0, 0)),
          pl.BlockSpec((DIM, NV_BLK), lambda i: (0, i)),
          pl.BlockSpec((1, NV_BLK), lambda i: (0, i)),
      ],
      out_specs=pl.BlockSpec((NV_BLK, B), lambda i: (i, 0)),
      out_shape=jax.ShapeDtypeStruct((V, B), jnp.float32),
      compiler_params=pltpu.CompilerParams(
          dimension_semantics=("parallel",),
      ),
  )(summed, w_t, b_row)


@jax.jit
def kernel(X, emb_table, W, b):
  xa = X.reshape(B * L)
  tbl128 = _tc_pack(emb_table.T)
  summed = _sc_gather_sum(xa, tbl128)
  out_t = _tc_project_t(summed, W.T, b.reshape(1, -1))
  return out_t.T
